# Initial kernel scaffold; baseline (speedup 1.0000x reference)
import jax
import jax.numpy as jnp
from jax.experimental import pallas as pl


def kernel(user, item, user_table, item_table, W1, b1, W2, b2):
    raise NotImplementedError("write your pallas kernel here")


# speedup vs baseline: 6.8900x; 6.8900x over previous
import functools

import jax
import jax.numpy as jnp
from jax import lax
from jax.experimental import pallas as pl
from jax.experimental.pallas import tpu as pltpu
from jax.experimental.pallas import tpu_sc as plsc

NC, NS = 2, 16
NW = NC * NS
B = 16384
D = 16
BPW = B // NW
CH = 128
NCH = BPW // CH


def _gather_body(user_hbm, item_hbm, ut_hbm, it_hbm,
                 urows_out, irows_out,
                 uidx_v, iidx_v, urows_v, irows_v, sem):
    wid = lax.axis_index("s") * NC + lax.axis_index("c")
    base = wid * BPW
    pltpu.sync_copy(user_hbm.at[pl.ds(base, BPW)], uidx_v)
    pltpu.sync_copy(item_hbm.at[pl.ds(base, BPW)], iidx_v)
    copies = []
    for j in range(NCH):
        sl = pl.ds(j * CH, CH)
        copies.append(pltpu.async_copy(ut_hbm.at[uidx_v.at[sl]],
                                       urows_v.at[sl], sem))
        copies.append(pltpu.async_copy(it_hbm.at[iidx_v.at[sl]],
                                       irows_v.at[sl], sem))
    for c in copies:
        c.wait()
    pltpu.sync_copy(urows_v, urows_out.at[pl.ds(base, BPW)])
    pltpu.sync_copy(irows_v, irows_out.at[pl.ds(base, BPW)])


def _sc_gather(user, item, user_table, item_table):
    mesh = plsc.VectorSubcoreMesh(core_axis_name="c", subcore_axis_name="s",
                                  num_cores=NC, num_subcores=NS)
    f = pl.kernel(
        _gather_body,
        out_type=(jax.ShapeDtypeStruct((B, D), jnp.float32),
                  jax.ShapeDtypeStruct((B, D), jnp.float32)),
        mesh=mesh,
        scratch_types=[
            pltpu.VMEM((BPW,), jnp.int32),
            pltpu.VMEM((BPW,), jnp.int32),
            pltpu.VMEM((BPW, D), jnp.float32),
            pltpu.VMEM((BPW, D), jnp.float32),
            pltpu.SemaphoreType.DMA,
        ],
    )
    return f(user, item, user_table, item_table)


def _mlp_body(u_ref, i_ref, w1u_ref, w1i_ref, b1_ref, w2_ref, b2_ref, o_ref):
    h = u_ref[...] @ w1u_ref[...] + i_ref[...] @ w1i_ref[...] + b1_ref[...]
    h = jnp.maximum(h, 0.0)
    o_ref[...] = jax.nn.sigmoid(h @ w2_ref[...] + b2_ref[...])


def _tc_mlp(u_rows, i_rows, W1, b1, W2, b2):
    w1u = W1[:D, :]
    w1i = W1[D:, :]
    b1r = b1.reshape(1, 16)
    b2r = b2.reshape(1, 1)
    grid = 8
    blk = B // grid
    return pl.pallas_call(
        _mlp_body,
        grid=(grid,),
        in_specs=[
            pl.BlockSpec((blk, D), lambda n: (n, 0)),
            pl.BlockSpec((blk, D), lambda n: (n, 0)),
            pl.BlockSpec((D, 16), lambda n: (0, 0)),
            pl.BlockSpec((D, 16), lambda n: (0, 0)),
            pl.BlockSpec((1, 16), lambda n: (0, 0)),
            pl.BlockSpec((16, 1), lambda n: (0, 0)),
            pl.BlockSpec((1, 1), lambda n: (0, 0)),
        ],
        out_specs=pl.BlockSpec((blk, 1), lambda n: (n, 0)),
        out_shape=jax.ShapeDtypeStruct((B, 1), jnp.float32),
    )(u_rows, i_rows, w1u, w1i, b1r, W2, b2r)


def kernel(user, item, user_table, item_table, W1, b1, W2, b2):
    u_rows = jnp.take(user_table, user, axis=0)
    i_rows = jnp.take(item_table, item, axis=0)
    return _tc_mlp(u_rows, i_rows, W1, b1, W2, b2)


# speedup vs baseline: 7.2675x; 1.0548x over previous
import jax
import jax.numpy as jnp
from jax.experimental import pallas as pl

B = 16384
D = 16
GRID = 8
BLK = B // GRID


def _mlp_body(u_ref, i_ref, w1u_ref, w1i_ref, b1_ref, w2_ref, b2_ref, o_ref):
    h = u_ref[...] @ w1u_ref[...] + i_ref[...] @ w1i_ref[...] + b1_ref[...]
    h = jnp.maximum(h, 0.0)
    o2 = h @ w2_ref[...] + b2_ref[...]
    o_ref[...] = jax.nn.sigmoid(o2[:, 0])


def _tc_mlp(u_rows, i_rows, W1, b1, W2, b2):
    w1u = W1[:D, :]
    w1i = W1[D:, :]
    b1r = b1.reshape(1, D)
    b2r = b2.reshape(1, 1)
    return pl.pallas_call(
        _mlp_body,
        grid=(GRID,),
        in_specs=[
            pl.BlockSpec((BLK, D), lambda n: (n, 0)),
            pl.BlockSpec((BLK, D), lambda n: (n, 0)),
            pl.BlockSpec((D, D), lambda n: (0, 0)),
            pl.BlockSpec((D, D), lambda n: (0, 0)),
            pl.BlockSpec((1, D), lambda n: (0, 0)),
            pl.BlockSpec((D, 1), lambda n: (0, 0)),
            pl.BlockSpec((1, 1), lambda n: (0, 0)),
        ],
        out_specs=pl.BlockSpec((BLK,), lambda n: (n,)),
        out_shape=jax.ShapeDtypeStruct((B,), jnp.float32),
    )(u_rows, i_rows, w1u, w1i, b1r, W2, b2r)


def kernel(user, item, user_table, item_table, W1, b1, W2, b2):
    u_rows = jnp.take(user_table, user, axis=0)
    i_rows = jnp.take(item_table, item, axis=0)
    out = _tc_mlp(u_rows, i_rows, W1, b1, W2, b2)
    return out.reshape(B, 1)


# speedup vs baseline: 9.8087x; 1.3497x over previous
import jax
import jax.numpy as jnp
from jax.experimental import pallas as pl

B = 16384
D = 16
GRID = 1
NBLK = B // GRID


def _mlp_body(uT_ref, iT_ref, wu_ref, wi_ref, b1_ref, w2_ref, b2_ref, o_ref):
    hT = (wu_ref[...] @ uT_ref[...] + wi_ref[...] @ iT_ref[...]
          + b1_ref[...])
    hT = jnp.maximum(hT, 0.0)
    oT = w2_ref[...] @ hT + b2_ref[...]
    o_ref[...] = jax.nn.sigmoid(oT)


def _tc_mlp(uT, iT, wuT, wiT, b1c, w2T, b2r):
    return pl.pallas_call(
        _mlp_body,
        grid=(GRID,),
        in_specs=[
            pl.BlockSpec((D, NBLK), lambda n: (0, n)),
            pl.BlockSpec((D, NBLK), lambda n: (0, n)),
            pl.BlockSpec((D, D), lambda n: (0, 0)),
            pl.BlockSpec((D, D), lambda n: (0, 0)),
            pl.BlockSpec((D, 1), lambda n: (0, 0)),
            pl.BlockSpec((1, D), lambda n: (0, 0)),
            pl.BlockSpec((1, 1), lambda n: (0, 0)),
        ],
        out_specs=pl.BlockSpec((1, NBLK), lambda n: (0, n)),
        out_shape=jax.ShapeDtypeStruct((1, B), jnp.float32),
    )(uT, iT, wuT, wiT, b1c, w2T, b2r)


def kernel(user, item, user_table, item_table, W1, b1, W2, b2):
    H = B // 2
    u0 = user_table.at[user[:H]].get(mode="promise_in_bounds")
    i0 = item_table.at[item[:H]].get(mode="promise_in_bounds")
    u1 = user_table.at[user[H:]].get(mode="promise_in_bounds")
    i1 = item_table.at[item[H:]].get(mode="promise_in_bounds")
    u_rows = jnp.concatenate([u0, u1], axis=0)
    i_rows = jnp.concatenate([i0, i1], axis=0)
    w1T = W1.T
    out = _tc_mlp(u_rows.T, i_rows.T,
                  w1T[:, :D], w1T[:, D:],
                  b1.reshape(D, 1), W2.T, b2.reshape(1, 1))
    return out.reshape(B, 1)


# speedup vs baseline: 12.6164x; 1.2862x over previous
import jax
import jax.numpy as jnp
from jax.experimental import pallas as pl

B = 16384
D = 16
GRID = 1
NBLK = B // GRID


def _mlp_body(uT_ref, iT_ref, wu_ref, wi_ref, b1_ref, w2_ref, b2_ref, o_ref):
    hT = (wu_ref[...] @ uT_ref[...] + wi_ref[...] @ iT_ref[...]
          + b1_ref[...])
    hT = jnp.maximum(hT, 0.0)
    oT = w2_ref[...] @ hT + b2_ref[...]
    o_ref[...] = jax.nn.sigmoid(oT)


def _tc_mlp(uT, iT, wuT, wiT, b1c, w2T, b2r):
    return pl.pallas_call(
        _mlp_body,
        grid=(GRID,),
        in_specs=[
            pl.BlockSpec((D, NBLK), lambda n: (0, n)),
            pl.BlockSpec((D, NBLK), lambda n: (0, n)),
            pl.BlockSpec((D, D), lambda n: (0, 0)),
            pl.BlockSpec((D, D), lambda n: (0, 0)),
            pl.BlockSpec((D, 1), lambda n: (0, 0)),
            pl.BlockSpec((1, D), lambda n: (0, 0)),
            pl.BlockSpec((1, 1), lambda n: (0, 0)),
        ],
        out_specs=pl.BlockSpec((1, NBLK), lambda n: (0, n)),
        out_shape=jax.ShapeDtypeStruct((1, B), jnp.float32),
    )(uT, iT, wuT, wiT, b1c, w2T, b2r)


def kernel(user, item, user_table, item_table, W1, b1, W2, b2):
    u_rows = user_table.at[user].get(mode="promise_in_bounds")
    i_rows = item_table.at[item].get(mode="promise_in_bounds")
    w1T = W1.T
    out = _tc_mlp(u_rows.T, i_rows.T,
                  w1T[:, :D], w1T[:, D:],
                  b1.reshape(D, 1), W2.T, b2.reshape(1, 1))
    return out.reshape(B, 1)
